# Initial kernel scaffold; baseline (speedup 1.0000x reference)
import jax
import jax.numpy as jnp
from jax.experimental import pallas as pl


def kernel(x, edge_index, W_self1, W_neigh1, b1, W_self2, W_neigh2, b2):
    raise NotImplementedError("write your pallas kernel here")


# speedup vs baseline: 3.3635x; 3.3635x over previous
import functools

import jax
import jax.numpy as jnp
from jax import lax
from jax.experimental import pallas as pl
from jax.experimental.pallas import tpu as pltpu
from jax.experimental.pallas import tpu_sc as plsc

NC = 2
NS = 16
NW = NC * NS
CH = 128
NNODE = 10000
FDIM = 128
N_PAD = 10240
RPS = N_PAD // NS


def _seg_sum_cnt_body(feat, src3, dst3, z2d, z1d, ones_h, out_p, out_c,
                      acc_sh, cnt_sh, src_v, dst_v, rows_v, ones_v, sem,
                      *, nch):
  c = lax.axis_index("c")
  s = lax.axis_index("s")
  wid = s * NC + c
  base = s * RPS
  pltpu.sync_copy(z2d.at[pl.ds(base, RPS)], acc_sh.at[pl.ds(base, RPS)])
  pltpu.sync_copy(z1d.at[pl.ds(base, RPS)], cnt_sh.at[pl.ds(base, RPS)])
  pltpu.sync_copy(ones_h, ones_v)
  pltpu.sync_copy(src3.at[wid], src_v)
  pltpu.sync_copy(dst3.at[wid], dst_v)
  plsc.subcore_barrier()

  def chunk(j, carry):
    pltpu.async_copy(feat.at[src_v.at[j]], rows_v, sem).wait()
    pltpu.sync_copy(rows_v, acc_sh.at[dst_v.at[j]], add=True)
    pltpu.sync_copy(ones_v, cnt_sh.at[dst_v.at[j]], add=True)
    return carry

  lax.fori_loop(0, nch, chunk, 0)
  plsc.subcore_barrier()
  pltpu.sync_copy(acc_sh.at[pl.ds(base, RPS)], out_p.at[c, pl.ds(base, RPS)])
  pltpu.sync_copy(cnt_sh.at[pl.ds(base, RPS)], out_c.at[c, pl.ds(base, RPS)])


def _seg_sum_body(feat, src3, dst3, z2d, out_p,
                  acc_sh, src_v, dst_v, rows_v, sem, *, nch):
  c = lax.axis_index("c")
  s = lax.axis_index("s")
  wid = s * NC + c
  base = s * RPS
  pltpu.sync_copy(z2d.at[pl.ds(base, RPS)], acc_sh.at[pl.ds(base, RPS)])
  pltpu.sync_copy(src3.at[wid], src_v)
  pltpu.sync_copy(dst3.at[wid], dst_v)
  plsc.subcore_barrier()

  def chunk(j, carry):
    pltpu.async_copy(feat.at[src_v.at[j]], rows_v, sem).wait()
    pltpu.sync_copy(rows_v, acc_sh.at[dst_v.at[j]], add=True)
    return carry

  lax.fori_loop(0, nch, chunk, 0)
  plsc.subcore_barrier()
  pltpu.sync_copy(acc_sh.at[pl.ds(base, RPS)], out_p.at[c, pl.ds(base, RPS)])


def _make_seg_kernels(nch):
  mesh = plsc.VectorSubcoreMesh(core_axis_name="c", subcore_axis_name="s")
  seg_cnt = pl.kernel(
      functools.partial(_seg_sum_cnt_body, nch=nch),
      out_type=(jax.ShapeDtypeStruct((NC, N_PAD, FDIM), jnp.float32),
                jax.ShapeDtypeStruct((NC, N_PAD), jnp.float32)),
      mesh=mesh,
      scratch_types=[
          pltpu.VMEM_SHARED((N_PAD, FDIM), jnp.float32),
          pltpu.VMEM_SHARED((N_PAD,), jnp.float32),
          pltpu.VMEM((nch, CH), jnp.int32),
          pltpu.VMEM((nch, CH), jnp.int32),
          pltpu.VMEM((CH, FDIM), jnp.float32),
          pltpu.VMEM((CH,), jnp.float32),
          pltpu.SemaphoreType.DMA,
      ],
      name="sage_seg_sum_cnt",
  )
  seg = pl.kernel(
      functools.partial(_seg_sum_body, nch=nch),
      out_type=jax.ShapeDtypeStruct((NC, N_PAD, FDIM), jnp.float32),
      mesh=mesh,
      scratch_types=[
          pltpu.VMEM_SHARED((N_PAD, FDIM), jnp.float32),
          pltpu.VMEM((nch, CH), jnp.int32),
          pltpu.VMEM((nch, CH), jnp.int32),
          pltpu.VMEM((CH, FDIM), jnp.float32),
          pltpu.SemaphoreType.DMA,
      ],
      name="sage_seg_sum",
  )
  return seg_cnt, seg


BR = 1000


def _dense_body(x_ref, p_ref, c_ref, ws_ref, wn_ref, b_ref, o_ref):
  p = p_ref[0] + p_ref[1]
  cnt = c_ref[0] + c_ref[1]
  agg = p / jnp.maximum(cnt, 1.0)
  acc = jnp.dot(x_ref[...], ws_ref[...], preferred_element_type=jnp.float32)
  acc = acc + jnp.dot(agg, wn_ref[...], preferred_element_type=jnp.float32)
  o_ref[...] = jnp.maximum(acc + b_ref[...], 0.0)


def _dense(x, p, cnt3, ws, wn, b):
  nb = NNODE // BR
  return pl.pallas_call(
      _dense_body,
      grid=(nb,),
      in_specs=[
          pl.BlockSpec((BR, FDIM), lambda i: (i, 0)),
          pl.BlockSpec((NC, BR, FDIM), lambda i: (0, i, 0)),
          pl.BlockSpec((NC, BR, 1), lambda i: (0, i, 0)),
          pl.BlockSpec((FDIM, FDIM), lambda i: (0, 0)),
          pl.BlockSpec((FDIM, FDIM), lambda i: (0, 0)),
          pl.BlockSpec((1, FDIM), lambda i: (0, 0)),
      ],
      out_specs=pl.BlockSpec((BR, FDIM), lambda i: (i, 0)),
      out_shape=jax.ShapeDtypeStruct((NNODE, FDIM), jnp.float32),
  )(x, p, cnt3, ws, wn, b.reshape(1, FDIM))


def kernel(x, edge_index, W_self1, W_neigh1, b1, W_self2, W_neigh2, b2):
  e = edge_index.shape[1]
  nch = -(-e // (NW * CH))
  nch += nch % 2
  e_pad = NW * nch * CH
  src = edge_index[0]
  dst = edge_index[1]
  pad = e_pad - e
  src3 = jnp.concatenate(
      [src, jnp.zeros((pad,), jnp.int32)]).reshape(NW, nch, CH)
  dst3 = jnp.concatenate(
      [dst, jnp.full((pad,), NNODE, jnp.int32)]).reshape(NW, nch, CH)
  z2d = jnp.zeros((N_PAD, FDIM), jnp.float32)
  z1d = jnp.zeros((N_PAD,), jnp.float32)
  ones_h = jnp.ones((CH,), jnp.float32)

  seg_cnt, seg = _make_seg_kernels(nch)
  p1, cnts = seg_cnt(x, src3, dst3, z2d, z1d, ones_h)
  cnt3 = cnts.reshape(NC, N_PAD, 1)
  h = _dense(x, p1, cnt3, W_self1, W_neigh1, b1)
  p2 = seg(h, src3, dst3, z2d)
  return _dense(h, p2, cnt3, W_self2, W_neigh2, b2)


# speedup vs baseline: 3.6894x; 1.0969x over previous
import functools

import jax
import jax.numpy as jnp
from jax import lax
from jax.experimental import pallas as pl
from jax.experimental.pallas import tpu as pltpu
from jax.experimental.pallas import tpu_sc as plsc

NC = 2
NS = 16
NW = NC * NS
CH = 128
NBUF = 2
NNODE = 10000
FDIM = 128
N_PAD = 10240
RPS = N_PAD // NS


def _seg_loop(with_cnt, nch, feat, wid, src3, dst_v, acc_sh, cnt_sh, ones_v,
              isrc, rows, isem, gsem, ssem, csem):
  ng = nch // NBUF

  def _idx(j, b):
    return pltpu.make_async_copy(src3.at[wid, j], isrc[b], isem.at[b])

  def _gather(b):
    return pltpu.make_async_copy(feat.at[isrc[b]], rows[b], gsem.at[b])

  def _scat(j, b):
    return pltpu.make_async_copy(rows[b], acc_sh.at[dst_v.at[j]], ssem.at[b])

  def _cnt(j, b):
    return pltpu.make_async_copy(ones_v, cnt_sh.at[dst_v.at[j]], csem.at[b])

  for b in range(NBUF):
    _idx(b, b).start()
    _idx(b, b).wait()
    _gather(b).start()

  def group(t, carry):
    base = t * NBUF
    for b in range(NBUF):
      j = base + b
      _gather(b).wait()
      _scat(j, b).start()
      if with_cnt:
        _cnt(j, b).start()
      _idx((j + NBUF) % nch, b).start()
    for b in range(NBUF):
      j = base + b
      _scat(j, b).wait()
      if with_cnt:
        _cnt(j, b).wait()
      _idx(j, b).wait()
      _gather(b).start()
    return carry

  lax.fori_loop(0, ng, group, 0)
  for b in range(NBUF):
    _gather(b).wait()


def _seg_sum_cnt_body(feat, src3, dst3, z2d, z1d, ones_h, out_p, out_c,
                      acc_sh, cnt_sh, dst_v, ones_v,
                      isrc0, isrc1, rows0, rows1, isem, gsem, ssem, csem,
                      *, nch):
  c = lax.axis_index("c")
  s = lax.axis_index("s")
  wid = s * NC + c
  base = s * RPS
  pltpu.sync_copy(z2d.at[pl.ds(base, RPS)], acc_sh.at[pl.ds(base, RPS)])
  pltpu.sync_copy(z1d.at[pl.ds(base, RPS)], cnt_sh.at[pl.ds(base, RPS)])
  pltpu.sync_copy(ones_h, ones_v)
  pltpu.sync_copy(dst3.at[wid], dst_v)
  plsc.subcore_barrier()
  _seg_loop(True, nch, feat, wid, src3, dst_v, acc_sh, cnt_sh, ones_v,
            (isrc0, isrc1), (rows0, rows1), isem, gsem, ssem, csem)
  plsc.subcore_barrier()
  pltpu.sync_copy(acc_sh.at[pl.ds(base, RPS)], out_p.at[c, pl.ds(base, RPS)])
  pltpu.sync_copy(cnt_sh.at[pl.ds(base, RPS)], out_c.at[c, pl.ds(base, RPS)])


def _seg_sum_body(feat, src3, dst3, z2d, out_p,
                  acc_sh, dst_v,
                  isrc0, isrc1, rows0, rows1, isem, gsem, ssem,
                  *, nch):
  c = lax.axis_index("c")
  s = lax.axis_index("s")
  wid = s * NC + c
  base = s * RPS
  pltpu.sync_copy(z2d.at[pl.ds(base, RPS)], acc_sh.at[pl.ds(base, RPS)])
  pltpu.sync_copy(dst3.at[wid], dst_v)
  plsc.subcore_barrier()
  _seg_loop(False, nch, feat, wid, src3, dst_v, acc_sh, None, None,
            (isrc0, isrc1), (rows0, rows1), isem, gsem, ssem, None)
  plsc.subcore_barrier()
  pltpu.sync_copy(acc_sh.at[pl.ds(base, RPS)], out_p.at[c, pl.ds(base, RPS)])


def _make_seg_kernels(nch):
  mesh = plsc.VectorSubcoreMesh(core_axis_name="c", subcore_axis_name="s")
  ring_bufs = [pltpu.VMEM((CH,), jnp.int32) for _ in range(NBUF)] + [
      pltpu.VMEM((CH, FDIM), jnp.float32) for _ in range(NBUF)]
  seg_cnt = pl.kernel(
      functools.partial(_seg_sum_cnt_body, nch=nch),
      out_type=(jax.ShapeDtypeStruct((NC, N_PAD, FDIM), jnp.float32),
                jax.ShapeDtypeStruct((NC, N_PAD), jnp.float32)),
      mesh=mesh,
      scratch_types=[
          pltpu.VMEM_SHARED((N_PAD, FDIM), jnp.float32),
          pltpu.VMEM_SHARED((N_PAD,), jnp.float32),
          pltpu.VMEM((nch, CH), jnp.int32),
          pltpu.VMEM((CH,), jnp.float32),
      ] + ring_bufs + [
          pltpu.SemaphoreType.DMA((NBUF,)),
          pltpu.SemaphoreType.DMA((NBUF,)),
          pltpu.SemaphoreType.DMA((NBUF,)),
          pltpu.SemaphoreType.DMA((NBUF,)),
      ],
      name="sage_seg_sum_cnt",
  )
  seg = pl.kernel(
      functools.partial(_seg_sum_body, nch=nch),
      out_type=jax.ShapeDtypeStruct((NC, N_PAD, FDIM), jnp.float32),
      mesh=mesh,
      scratch_types=[
          pltpu.VMEM_SHARED((N_PAD, FDIM), jnp.float32),
          pltpu.VMEM((nch, CH), jnp.int32),
      ] + ring_bufs + [
          pltpu.SemaphoreType.DMA((NBUF,)),
          pltpu.SemaphoreType.DMA((NBUF,)),
          pltpu.SemaphoreType.DMA((NBUF,)),
      ],
      name="sage_seg_sum",
  )
  return seg_cnt, seg


BR = 1000


def _dense_body(x_ref, p_ref, c_ref, ws_ref, wn_ref, b_ref, o_ref):
  p = p_ref[0] + p_ref[1]
  cnt = c_ref[0] + c_ref[1]
  agg = p / jnp.maximum(cnt, 1.0)
  acc = jnp.dot(x_ref[...], ws_ref[...], preferred_element_type=jnp.float32)
  acc = acc + jnp.dot(agg, wn_ref[...], preferred_element_type=jnp.float32)
  o_ref[...] = jnp.maximum(acc + b_ref[...], 0.0)


def _dense(x, p, cnt3, ws, wn, b):
  nb = NNODE // BR
  return pl.pallas_call(
      _dense_body,
      grid=(nb,),
      in_specs=[
          pl.BlockSpec((BR, FDIM), lambda i: (i, 0)),
          pl.BlockSpec((NC, BR, FDIM), lambda i: (0, i, 0)),
          pl.BlockSpec((NC, BR, 1), lambda i: (0, i, 0)),
          pl.BlockSpec((FDIM, FDIM), lambda i: (0, 0)),
          pl.BlockSpec((FDIM, FDIM), lambda i: (0, 0)),
          pl.BlockSpec((1, FDIM), lambda i: (0, 0)),
      ],
      out_specs=pl.BlockSpec((BR, FDIM), lambda i: (i, 0)),
      out_shape=jax.ShapeDtypeStruct((NNODE, FDIM), jnp.float32),
  )(x, p, cnt3, ws, wn, b.reshape(1, FDIM))


def kernel(x, edge_index, W_self1, W_neigh1, b1, W_self2, W_neigh2, b2):
  e = edge_index.shape[1]
  nch = -(-e // (NW * CH))
  nch = -(-nch // NBUF) * NBUF
  e_pad = NW * nch * CH
  src = edge_index[0]
  dst = edge_index[1]
  pad = e_pad - e
  src3 = jnp.concatenate(
      [src, jnp.zeros((pad,), jnp.int32)]).reshape(NW, nch, CH)
  dst3 = jnp.concatenate(
      [dst, jnp.full((pad,), NNODE, jnp.int32)]).reshape(NW, nch, CH)
  z2d = jnp.zeros((N_PAD, FDIM), jnp.float32)
  z1d = jnp.zeros((N_PAD,), jnp.float32)
  ones_h = jnp.ones((CH,), jnp.float32)

  seg_cnt, seg = _make_seg_kernels(nch)
  p1, cnts = seg_cnt(x, src3, dst3, z2d, z1d, ones_h)
  cnt3 = cnts.reshape(NC, N_PAD, 1)
  h = _dense(x, p1, cnt3, W_self1, W_neigh1, b1)
  p2 = seg(h, src3, dst3, z2d)
  return _dense(h, p2, cnt3, W_self2, W_neigh2, b2)
